# baseline (speedup 1.0000x reference)
import functools

import jax
import jax.numpy as jnp
from jax import lax
from jax.experimental import pallas as pl
from jax.experimental.pallas import tpu as pltpu
from jax.experimental.pallas import tpu_sc as plsc

N = 10000
E = 320000
D = 128
NPAD = 10240

NC = 1
NS = 16
NW = NC * NS
K = 80
CHUNKS = E // K // NW
CB = 10
NBLK = CHUNKS // CB
ROWS_PER_SUB = NPAD // NS
NBUF = 2

_mesh = plsc.VectorSubcoreMesh(core_axis_name="c", subcore_axis_name="s",
                               num_cores=NC)


@functools.partial(
    pl.kernel,
    out_type=jax.ShapeDtypeStruct((NC, NPAD, D), jnp.float32),
    mesh=_mesh,
    scratch_types=[
        pltpu.VMEM((2, CB, K), jnp.int32),
        pltpu.VMEM((2, CB, K), jnp.int32),
        [pltpu.VMEM((K, D), jnp.float32) for _ in range(NBUF)],
        pltpu.VMEM_SHARED((NPAD, D), jnp.float32),
        [pltpu.SemaphoreType.DMA for _ in range(NBUF)],
        pltpu.SemaphoreType.DMA,
        pltpu.SemaphoreType.DMA,
    ],
)
def _sc_scatter(h_hbm, row_hbm, col_hbm, zeros_hbm, out_hbm,
                rowbuf, colbuf, rows_v, agg_sh, sems, isem_r, isem_c):
    c = lax.axis_index("c")
    s = lax.axis_index("s")
    w = c * NS + s

    pltpu.sync_copy(zeros_hbm, agg_sh.at[pl.ds(s * ROWS_PER_SUB, ROWS_PER_SUB)])
    plsc.subcore_barrier()

    def fire(slot, j, b):
        pltpu.async_copy(h_hbm.at[colbuf.at[slot, j]], rows_v[b], sems[b])

    def drain_scatter(slot, j, b):
        pltpu.make_async_copy(h_hbm.at[colbuf.at[slot, j]], rows_v[b],
                              sems[b]).wait()
        pltpu.sync_copy(rows_v[b], agg_sh.at[rowbuf.at[slot, j]], add=True)

    def refill_async(slot, blk):
        base = w * NBLK + blk
        pltpu.async_copy(row_hbm.at[pl.ds(base, 1)], rowbuf.at[pl.ds(slot, 1)],
                         isem_r)
        pltpu.async_copy(col_hbm.at[pl.ds(base, 1)], colbuf.at[pl.ds(slot, 1)],
                         isem_c)

    def refill_wait(slot, blk):
        base = w * NBLK + blk
        pltpu.make_async_copy(row_hbm.at[pl.ds(base, 1)],
                              rowbuf.at[pl.ds(slot, 1)], isem_r).wait()
        pltpu.make_async_copy(col_hbm.at[pl.ds(base, 1)],
                              colbuf.at[pl.ds(slot, 1)], isem_c).wait()

    pltpu.sync_copy(row_hbm.at[pl.ds(w * NBLK, 1)], rowbuf.at[pl.ds(0, 1)])
    pltpu.sync_copy(col_hbm.at[pl.ds(w * NBLK, 1)], colbuf.at[pl.ds(0, 1)])
    refill_async(1, 1)
    for b in range(NBUF):
        fire(0, b, b)

    def blk_body(blk, carry):
        slot = lax.rem(blk, 2)
        nslot = lax.rem(blk + 1, 2)
        for j in range(CB):
            if j == CB - NBUF:
                refill_wait(nslot, blk + 1)
            drain_scatter(slot, j, j % NBUF)
            if j < CB - NBUF:
                fire(slot, j + NBUF, j % NBUF)
            else:
                fire(nslot, j + NBUF - CB, j % NBUF)
        refill_async(slot, blk + 2)
        return carry

    lax.fori_loop(0, NBLK - 1, blk_body, 0)

    lslot = (NBLK - 1) % 2
    for j in range(CB):
        drain_scatter(lslot, j, j % NBUF)
        if j < CB - NBUF:
            fire(lslot, j + NBUF, j % NBUF)
    refill_wait(NBLK % 2, NBLK)
    plsc.subcore_barrier()

    pltpu.sync_copy(agg_sh.at[pl.ds(s * ROWS_PER_SUB, ROWS_PER_SUB)],
                    out_hbm.at[c, pl.ds(s * ROWS_PER_SUB, ROWS_PER_SUB)])


def _dense_body(aggp_ref, deg_ref, w_ref, b_ref, g_ref, be_ref, sbn_ref,
                out_ref):
    deg = deg_ref[:, 0] + 1e-6
    agg = sum(aggp_ref[i, :N] for i in range(NC)) * (1.0 / deg)[:, None]
    u = lax.dot_general(agg, w_ref[...], (((1,), (1,)), ((), ())),
                        preferred_element_type=jnp.float32) + b_ref[...]
    mean = jnp.mean(u, axis=0)
    var = jnp.mean(u * u, axis=0) - mean * mean
    h = (u - mean) * jax.lax.rsqrt(var + 1e-5) * g_ref[...] + be_ref[...]
    h = jnp.maximum(h, 0.0)
    f = sbn_ref[0, 0]
    out_ref[...] = f * h + (1.0 - f) * u


_dense = pl.pallas_call(
    _dense_body, out_shape=jax.ShapeDtypeStruct((N, D), jnp.float32))


def kernel(x, edge_index, W1, b1, g1, be1, W2, b2, g2, be2, W3, b3):
    pad = jnp.zeros((1, CB, K), jnp.int32)
    row = jnp.concatenate([edge_index[0].reshape(NW * NBLK, CB, K), pad])
    col = jnp.concatenate([edge_index[1].reshape(NW * NBLK, CB, K), pad])
    zeros_feat = jnp.zeros((ROWS_PER_SUB, D), jnp.float32)
    ones_feat = jnp.ones((N, D), jnp.float32)

    onesD = jnp.ones((D,), jnp.float32)
    zerosD = jnp.zeros((D,), jnp.float32)
    col0 = jnp.tile(jnp.arange(K, dtype=jnp.int32), (NW * NBLK + 1, CB, 1))
    cols_s = jnp.stack([col0, col, col, col])
    Ws = jnp.stack([W1, W1, W2, W3])
    bs = jnp.stack([zerosD, b1, b2, b3])
    gs = jnp.stack([onesD, g1, g2, onesD])
    bes = jnp.stack([zerosD, be1, be2, zerosD])
    s_bn = jnp.array([1.0, 1.0, 1.0, 0.0], jnp.float32).reshape(4, 1, 1)
    s_deg = jnp.array([1.0, 0.0, 0.0, 0.0], jnp.float32)

    def body(carry, xs):
        h, deg = carry
        cols_i, W, b, g, be, f_bn, f_deg = xs
        aggp = _sc_scatter(h, row, cols_i, zeros_feat)
        deg_new = sum(aggp[i, :N, :1] for i in range(NC))
        deg = jnp.where(f_deg > 0, deg_new, deg)
        out = _dense(aggp, deg, W, b, g, be, f_bn)
        h = jnp.where(f_deg > 0, x, out)
        return (h, deg), None

    init = (ones_feat, jnp.ones((N, 1), jnp.float32))
    (h, _), _ = lax.scan(body, init, (cols_s, Ws, bs, gs, bes, s_bn, s_deg))
    return h


# speedup vs baseline: 1.0955x; 1.0955x over previous
import functools

import jax
import jax.numpy as jnp
from jax import lax
from jax.experimental import pallas as pl
from jax.experimental.pallas import tpu as pltpu
from jax.experimental.pallas import tpu_sc as plsc

N = 10000
E = 320000
D = 128
NPAD = 10240

NC = 1
NS = 16
NW = NC * NS
K = 80
CHUNKS = E // K // NW
CB = 10
NBLK = CHUNKS // CB
ROWS_PER_SUB = NPAD // NS
NBUF = 2

_mesh = plsc.VectorSubcoreMesh(core_axis_name="c", subcore_axis_name="s",
                               num_cores=NC)


@functools.partial(
    pl.kernel,
    out_type=jax.ShapeDtypeStruct((NC, NPAD, D), jnp.float32),
    mesh=_mesh,
    scratch_types=[
        pltpu.VMEM((2, CB, K), jnp.int32),
        pltpu.VMEM((2, CB, K), jnp.int32),
        [pltpu.VMEM((K, D), jnp.float32) for _ in range(NBUF)],
        pltpu.VMEM_SHARED((NPAD, D), jnp.float32),
        [pltpu.SemaphoreType.DMA for _ in range(NBUF)],
        pltpu.SemaphoreType.DMA,
        pltpu.SemaphoreType.DMA,
    ],
)
def _sc_scatter(h_hbm, row_hbm, col_hbm, zeros_hbm, out_hbm,
                rowbuf, colbuf, rows_v, agg_sh, sems, isem_r, isem_c):
    c = lax.axis_index("c")
    s = lax.axis_index("s")
    w = c * NS + s

    pltpu.sync_copy(zeros_hbm, agg_sh.at[pl.ds(s * ROWS_PER_SUB, ROWS_PER_SUB)])
    plsc.subcore_barrier()

    def fire(slot, j, b):
        pltpu.async_copy(h_hbm.at[colbuf.at[slot, j]], rows_v[b], sems[b])

    def drain_scatter(slot, j, b):
        pltpu.make_async_copy(h_hbm.at[colbuf.at[slot, j]], rows_v[b],
                              sems[b]).wait()
        pltpu.sync_copy(rows_v[b], agg_sh.at[rowbuf.at[slot, j]], add=True)

    def refill_async(slot, blk):
        base = w * NBLK + blk
        pltpu.async_copy(row_hbm.at[pl.ds(base, 1)], rowbuf.at[pl.ds(slot, 1)],
                         isem_r)
        pltpu.async_copy(col_hbm.at[pl.ds(base, 1)], colbuf.at[pl.ds(slot, 1)],
                         isem_c)

    def refill_wait(slot, blk):
        base = w * NBLK + blk
        pltpu.make_async_copy(row_hbm.at[pl.ds(base, 1)],
                              rowbuf.at[pl.ds(slot, 1)], isem_r).wait()
        pltpu.make_async_copy(col_hbm.at[pl.ds(base, 1)],
                              colbuf.at[pl.ds(slot, 1)], isem_c).wait()

    pltpu.sync_copy(row_hbm.at[pl.ds(w * NBLK, 1)], rowbuf.at[pl.ds(0, 1)])
    pltpu.sync_copy(col_hbm.at[pl.ds(w * NBLK, 1)], colbuf.at[pl.ds(0, 1)])
    refill_async(1, 1)
    for b in range(NBUF):
        fire(0, b, b)

    def blk_body(blk, carry):
        slot = lax.rem(blk, 2)
        nslot = lax.rem(blk + 1, 2)
        for j in range(CB):
            if j == CB - NBUF:
                refill_wait(nslot, blk + 1)
            drain_scatter(slot, j, j % NBUF)
            if j < CB - NBUF:
                fire(slot, j + NBUF, j % NBUF)
            else:
                fire(nslot, j + NBUF - CB, j % NBUF)
        refill_async(slot, blk + 2)
        return carry

    lax.fori_loop(0, NBLK - 1, blk_body, 0)

    lslot = (NBLK - 1) % 2
    for j in range(CB):
        drain_scatter(lslot, j, j % NBUF)
        if j < CB - NBUF:
            fire(lslot, j + NBUF, j % NBUF)
    refill_wait(NBLK % 2, NBLK)
    plsc.subcore_barrier()

    pltpu.sync_copy(agg_sh.at[pl.ds(s * ROWS_PER_SUB, ROWS_PER_SUB)],
                    out_hbm.at[c, pl.ds(s * ROWS_PER_SUB, ROWS_PER_SUB)])


def _dense_body(aggp_ref, deg_ref, w_ref, b_ref, g_ref, be_ref, sbn_ref,
                out_ref):
    deg = deg_ref[:, 0] + 1e-6
    agg = sum(aggp_ref[i, :N] for i in range(NC)) * (1.0 / deg)[:, None]
    u = lax.dot_general(agg, w_ref[...], (((1,), (1,)), ((), ())),
                        preferred_element_type=jnp.float32) + b_ref[...]
    mean = jnp.mean(u, axis=0)
    var = jnp.mean(u * u, axis=0) - mean * mean
    h = (u - mean) * jax.lax.rsqrt(var + 1e-5) * g_ref[...] + be_ref[...]
    h = jnp.maximum(h, 0.0)
    f = sbn_ref[0, 0]
    out_ref[...] = f * h + (1.0 - f) * u


_dense = pl.pallas_call(
    _dense_body, out_shape=jax.ShapeDtypeStruct((N, D), jnp.float32))


def kernel(x, edge_index, W1, b1, g1, be1, W2, b2, g2, be2, W3, b3):
    pad = jnp.zeros((1, CB, K), jnp.int32)
    row = jnp.concatenate([edge_index[0].reshape(NW * NBLK, CB, K), pad])
    col = jnp.concatenate([edge_index[1].reshape(NW * NBLK, CB, K), pad])
    zeros_feat = jnp.zeros((ROWS_PER_SUB, D), jnp.float32)
    ones_feat = jnp.ones((N, D), jnp.float32)

    onesD = jnp.ones((D,), jnp.float32)
    zerosD = jnp.zeros((D,), jnp.float32)
    cols_s = jnp.stack([row, col, col, col])
    Ws = jnp.stack([W1, W1, W2, W3])
    bs = jnp.stack([zerosD, b1, b2, b3])
    gs = jnp.stack([onesD, g1, g2, onesD])
    bes = jnp.stack([zerosD, be1, be2, zerosD])
    s_bn = jnp.array([1.0, 1.0, 1.0, 0.0], jnp.float32).reshape(4, 1, 1)
    s_deg = jnp.array([1.0, 0.0, 0.0, 0.0], jnp.float32)

    def body(carry, xs):
        h, deg = carry
        cols_i, W, b, g, be, f_bn, f_deg = xs
        aggp = _sc_scatter(h, row, cols_i, zeros_feat)
        deg_new = sum(aggp[i, :N, :1] for i in range(NC))
        deg = jnp.where(f_deg > 0, deg_new, deg)
        out = _dense(aggp, deg, W, b, g, be, f_bn)
        h = jnp.where(f_deg > 0, x, out)
        return (h, deg), None

    init = (ones_feat, jnp.ones((N, 1), jnp.float32))
    (h, _), _ = lax.scan(body, init, (cols_s, Ws, bs, gs, bes, s_bn, s_deg))
    return h


# speedup vs baseline: 1.2222x; 1.1157x over previous
import functools

import jax
import jax.numpy as jnp
from jax import lax
from jax.experimental import pallas as pl
from jax.experimental.pallas import tpu as pltpu
from jax.experimental.pallas import tpu_sc as plsc

N = 10000
E = 320000
D = 128
NPAD = 10240

NC = 1
NS = 16
NW = NC * NS
K = 40
CHUNKS = E // K // NW
CB = 20
NBLK = CHUNKS // CB
ROWS_PER_SUB = NPAD // NS
NBUF = 4

_mesh = plsc.VectorSubcoreMesh(core_axis_name="c", subcore_axis_name="s",
                               num_cores=NC)


@functools.partial(
    pl.kernel,
    out_type=jax.ShapeDtypeStruct((NC, NPAD, D), jnp.float32),
    mesh=_mesh,
    scratch_types=[
        pltpu.VMEM((2, CB, K), jnp.int32),
        pltpu.VMEM((2, CB, K), jnp.int32),
        [pltpu.VMEM((K, D), jnp.float32) for _ in range(NBUF)],
        pltpu.VMEM_SHARED((NPAD, D), jnp.float32),
        [pltpu.SemaphoreType.DMA for _ in range(NBUF)],
        pltpu.SemaphoreType.DMA,
        pltpu.SemaphoreType.DMA,
    ],
)
def _sc_scatter(h_hbm, row_hbm, col_hbm, zeros_hbm, out_hbm,
                rowbuf, colbuf, rows_v, agg_sh, sems, isem_r, isem_c):
    c = lax.axis_index("c")
    s = lax.axis_index("s")
    w = c * NS + s

    pltpu.sync_copy(zeros_hbm, agg_sh.at[pl.ds(s * ROWS_PER_SUB, ROWS_PER_SUB)])
    plsc.subcore_barrier()

    def fire(slot, j, b):
        pltpu.async_copy(h_hbm.at[colbuf.at[slot, j]], rows_v[b], sems[b])

    def drain_scatter(slot, j, b):
        pltpu.make_async_copy(h_hbm.at[colbuf.at[slot, j]], rows_v[b],
                              sems[b]).wait()
        pltpu.sync_copy(rows_v[b], agg_sh.at[rowbuf.at[slot, j]], add=True)

    def refill_async(slot, blk):
        base = w * NBLK + blk
        pltpu.async_copy(row_hbm.at[pl.ds(base, 1)], rowbuf.at[pl.ds(slot, 1)],
                         isem_r)
        pltpu.async_copy(col_hbm.at[pl.ds(base, 1)], colbuf.at[pl.ds(slot, 1)],
                         isem_c)

    def refill_wait(slot, blk):
        base = w * NBLK + blk
        pltpu.make_async_copy(row_hbm.at[pl.ds(base, 1)],
                              rowbuf.at[pl.ds(slot, 1)], isem_r).wait()
        pltpu.make_async_copy(col_hbm.at[pl.ds(base, 1)],
                              colbuf.at[pl.ds(slot, 1)], isem_c).wait()

    pltpu.sync_copy(row_hbm.at[pl.ds(w * NBLK, 1)], rowbuf.at[pl.ds(0, 1)])
    pltpu.sync_copy(col_hbm.at[pl.ds(w * NBLK, 1)], colbuf.at[pl.ds(0, 1)])
    refill_async(1, 1)
    for b in range(NBUF):
        fire(0, b, b)

    def blk_body(blk, carry):
        slot = lax.rem(blk, 2)
        nslot = lax.rem(blk + 1, 2)
        for j in range(CB):
            if j == CB - NBUF:
                refill_wait(nslot, blk + 1)
            drain_scatter(slot, j, j % NBUF)
            if j < CB - NBUF:
                fire(slot, j + NBUF, j % NBUF)
            else:
                fire(nslot, j + NBUF - CB, j % NBUF)
        refill_async(slot, blk + 2)
        return carry

    lax.fori_loop(0, NBLK - 1, blk_body, 0)

    lslot = (NBLK - 1) % 2
    for j in range(CB):
        drain_scatter(lslot, j, j % NBUF)
        if j < CB - NBUF:
            fire(lslot, j + NBUF, j % NBUF)
    refill_wait(NBLK % 2, NBLK)
    plsc.subcore_barrier()

    pltpu.sync_copy(agg_sh.at[pl.ds(s * ROWS_PER_SUB, ROWS_PER_SUB)],
                    out_hbm.at[c, pl.ds(s * ROWS_PER_SUB, ROWS_PER_SUB)])


def _dense_body(aggp_ref, deg_ref, w_ref, b_ref, g_ref, be_ref, sbn_ref,
                out_ref):
    deg = deg_ref[:, 0] + 1e-6
    agg = sum(aggp_ref[i, :N] for i in range(NC)) * (1.0 / deg)[:, None]
    u = lax.dot_general(agg, w_ref[...], (((1,), (1,)), ((), ())),
                        preferred_element_type=jnp.float32) + b_ref[...]
    mean = jnp.mean(u, axis=0)
    var = jnp.mean(u * u, axis=0) - mean * mean
    h = (u - mean) * jax.lax.rsqrt(var + 1e-5) * g_ref[...] + be_ref[...]
    h = jnp.maximum(h, 0.0)
    f = sbn_ref[0, 0]
    out_ref[...] = f * h + (1.0 - f) * u


_dense = pl.pallas_call(
    _dense_body, out_shape=jax.ShapeDtypeStruct((N, D), jnp.float32))


def kernel(x, edge_index, W1, b1, g1, be1, W2, b2, g2, be2, W3, b3):
    pad = jnp.zeros((1, CB, K), jnp.int32)
    row = jnp.concatenate([edge_index[0].reshape(NW * NBLK, CB, K), pad])
    col = jnp.concatenate([edge_index[1].reshape(NW * NBLK, CB, K), pad])
    zeros_feat = jnp.zeros((ROWS_PER_SUB, D), jnp.float32)
    ones_feat = jnp.ones((N, D), jnp.float32)

    onesD = jnp.ones((D,), jnp.float32)
    zerosD = jnp.zeros((D,), jnp.float32)
    cols_s = jnp.stack([row, col, col, col])
    Ws = jnp.stack([W1, W1, W2, W3])
    bs = jnp.stack([zerosD, b1, b2, b3])
    gs = jnp.stack([onesD, g1, g2, onesD])
    bes = jnp.stack([zerosD, be1, be2, zerosD])
    s_bn = jnp.array([1.0, 1.0, 1.0, 0.0], jnp.float32).reshape(4, 1, 1)
    s_deg = jnp.array([1.0, 0.0, 0.0, 0.0], jnp.float32)

    def body(carry, xs):
        h, deg = carry
        cols_i, W, b, g, be, f_bn, f_deg = xs
        aggp = _sc_scatter(h, row, cols_i, zeros_feat)
        deg_new = sum(aggp[i, :N, :1] for i in range(NC))
        deg = jnp.where(f_deg > 0, deg_new, deg)
        out = _dense(aggp, deg, W, b, g, be, f_bn)
        h = jnp.where(f_deg > 0, x, out)
        return (h, deg), None

    init = (ones_feat, jnp.ones((N, 1), jnp.float32))
    (h, _), _ = lax.scan(body, init, (cols_s, Ws, bs, gs, bes, s_bn, s_deg))
    return h


# speedup vs baseline: 1.2578x; 1.0291x over previous
import functools

import jax
import jax.numpy as jnp
from jax import lax
from jax.experimental import pallas as pl
from jax.experimental.pallas import tpu as pltpu
from jax.experimental.pallas import tpu_sc as plsc

N = 10000
E = 320000
D = 128
NPAD = 10240

NC = 1
NS = 16
NW = NC * NS
K = 32
CHUNKS = E // K // NW
CB = 25
NBLK = CHUNKS // CB
ROWS_PER_SUB = NPAD // NS
NBUF = 5

_mesh = plsc.VectorSubcoreMesh(core_axis_name="c", subcore_axis_name="s",
                               num_cores=NC)


@functools.partial(
    pl.kernel,
    out_type=jax.ShapeDtypeStruct((NC, NPAD, D), jnp.float32),
    mesh=_mesh,
    scratch_types=[
        pltpu.VMEM((2, CB, K), jnp.int32),
        pltpu.VMEM((2, CB, K), jnp.int32),
        [pltpu.VMEM((K, D), jnp.float32) for _ in range(NBUF)],
        pltpu.VMEM_SHARED((NPAD, D), jnp.float32),
        [pltpu.SemaphoreType.DMA for _ in range(NBUF)],
        pltpu.SemaphoreType.DMA,
        pltpu.SemaphoreType.DMA,
    ],
)
def _sc_scatter(h_hbm, row_hbm, col_hbm, zeros_hbm, out_hbm,
                rowbuf, colbuf, rows_v, agg_sh, sems, isem_r, isem_c):
    c = lax.axis_index("c")
    s = lax.axis_index("s")
    w = c * NS + s

    pltpu.sync_copy(zeros_hbm, agg_sh.at[pl.ds(s * ROWS_PER_SUB, ROWS_PER_SUB)])
    plsc.subcore_barrier()

    def fire(slot, j, b):
        pltpu.async_copy(h_hbm.at[colbuf.at[slot, j]], rows_v[b], sems[b])

    def drain_scatter(slot, j, b):
        pltpu.make_async_copy(h_hbm.at[colbuf.at[slot, j]], rows_v[b],
                              sems[b]).wait()
        pltpu.sync_copy(rows_v[b], agg_sh.at[rowbuf.at[slot, j]], add=True)

    def refill_async(slot, blk):
        base = w * NBLK + blk
        pltpu.async_copy(row_hbm.at[pl.ds(base, 1)], rowbuf.at[pl.ds(slot, 1)],
                         isem_r)
        pltpu.async_copy(col_hbm.at[pl.ds(base, 1)], colbuf.at[pl.ds(slot, 1)],
                         isem_c)

    def refill_wait(slot, blk):
        base = w * NBLK + blk
        pltpu.make_async_copy(row_hbm.at[pl.ds(base, 1)],
                              rowbuf.at[pl.ds(slot, 1)], isem_r).wait()
        pltpu.make_async_copy(col_hbm.at[pl.ds(base, 1)],
                              colbuf.at[pl.ds(slot, 1)], isem_c).wait()

    pltpu.sync_copy(row_hbm.at[pl.ds(w * NBLK, 1)], rowbuf.at[pl.ds(0, 1)])
    pltpu.sync_copy(col_hbm.at[pl.ds(w * NBLK, 1)], colbuf.at[pl.ds(0, 1)])
    refill_async(1, 1)
    for b in range(NBUF):
        fire(0, b, b)

    def blk_body(blk, carry):
        slot = lax.rem(blk, 2)
        nslot = lax.rem(blk + 1, 2)
        for j in range(CB):
            if j == CB - NBUF:
                refill_wait(nslot, blk + 1)
            drain_scatter(slot, j, j % NBUF)
            if j < CB - NBUF:
                fire(slot, j + NBUF, j % NBUF)
            else:
                fire(nslot, j + NBUF - CB, j % NBUF)
        refill_async(slot, blk + 2)
        return carry

    lax.fori_loop(0, NBLK - 1, blk_body, 0)

    lslot = (NBLK - 1) % 2
    for j in range(CB):
        drain_scatter(lslot, j, j % NBUF)
        if j < CB - NBUF:
            fire(lslot, j + NBUF, j % NBUF)
    refill_wait(NBLK % 2, NBLK)
    plsc.subcore_barrier()

    pltpu.sync_copy(agg_sh.at[pl.ds(s * ROWS_PER_SUB, ROWS_PER_SUB)],
                    out_hbm.at[c, pl.ds(s * ROWS_PER_SUB, ROWS_PER_SUB)])


def _dense_body(aggp_ref, deg_ref, w_ref, b_ref, g_ref, be_ref, sbn_ref,
                out_ref):
    deg = deg_ref[:, 0] + 1e-6
    agg = sum(aggp_ref[i, :N] for i in range(NC)) * (1.0 / deg)[:, None]
    u = lax.dot_general(agg, w_ref[...], (((1,), (1,)), ((), ())),
                        preferred_element_type=jnp.float32) + b_ref[...]
    mean = jnp.mean(u, axis=0)
    var = jnp.mean(u * u, axis=0) - mean * mean
    h = (u - mean) * jax.lax.rsqrt(var + 1e-5) * g_ref[...] + be_ref[...]
    h = jnp.maximum(h, 0.0)
    f = sbn_ref[0, 0]
    out_ref[...] = f * h + (1.0 - f) * u


_dense = pl.pallas_call(
    _dense_body, out_shape=jax.ShapeDtypeStruct((N, D), jnp.float32))


def kernel(x, edge_index, W1, b1, g1, be1, W2, b2, g2, be2, W3, b3):
    pad = jnp.zeros((1, CB, K), jnp.int32)
    row = jnp.concatenate([edge_index[0].reshape(NW * NBLK, CB, K), pad])
    col = jnp.concatenate([edge_index[1].reshape(NW * NBLK, CB, K), pad])
    zeros_feat = jnp.zeros((ROWS_PER_SUB, D), jnp.float32)
    ones_feat = jnp.ones((N, D), jnp.float32)

    onesD = jnp.ones((D,), jnp.float32)
    zerosD = jnp.zeros((D,), jnp.float32)
    cols_s = jnp.stack([row, col, col, col])
    Ws = jnp.stack([W1, W1, W2, W3])
    bs = jnp.stack([zerosD, b1, b2, b3])
    gs = jnp.stack([onesD, g1, g2, onesD])
    bes = jnp.stack([zerosD, be1, be2, zerosD])
    s_bn = jnp.array([1.0, 1.0, 1.0, 0.0], jnp.float32).reshape(4, 1, 1)
    s_deg = jnp.array([1.0, 0.0, 0.0, 0.0], jnp.float32)

    def body(carry, xs):
        h, deg = carry
        cols_i, W, b, g, be, f_bn, f_deg = xs
        aggp = _sc_scatter(h, row, cols_i, zeros_feat)
        deg_new = sum(aggp[i, :N, :1] for i in range(NC))
        deg = jnp.where(f_deg > 0, deg_new, deg)
        out = _dense(aggp, deg, W, b, g, be, f_bn)
        h = jnp.where(f_deg > 0, x, out)
        return (h, deg), None

    init = (ones_feat, jnp.ones((N, 1), jnp.float32))
    (h, _), _ = lax.scan(body, init, (cols_s, Ws, bs, gs, bes, s_bn, s_deg))
    return h
